# baseline (speedup 1.0000x reference)
import functools

import jax
import jax.numpy as jnp
from jax import lax
from jax.experimental import pallas as pl
from jax.experimental.pallas import tpu as pltpu
from jax.experimental.pallas import tpu_sc as plsc

N_NODES_S = 5000
N_PAD = 5120
N_EDGES_S = 20000
HEADS = 4
NW = 32


def _mm2_body(h_ref, wl_ref, bl_ref, wr_ref, br_ref, xl_ref, xr_ref):
    h = h_ref[...]
    xl_ref[...] = (
        jnp.dot(h, wl_ref[...], preferred_element_type=jnp.float32)
        + bl_ref[0:1, :]
    )
    xr_ref[...] = (
        jnp.dot(h, wr_ref[...], preferred_element_type=jnp.float32)
        + br_ref[0:1, :]
    )


@functools.partial(jax.jit, static_argnames=("bm", "bn"))
def _dual_matmul(h, wl, bl8, wr, br8, bm=512, bn=512):
    np_, k = h.shape
    hcp = wl.shape[1]
    bn = min(bn, hcp)
    grid = (np_ // bm, hcp // bn)
    return pl.pallas_call(
        _mm2_body,
        grid=grid,
        in_specs=[
            pl.BlockSpec((bm, k), lambda i, j: (i, 0)),
            pl.BlockSpec((k, bn), lambda i, j: (0, j)),
            pl.BlockSpec((8, bn), lambda i, j: (0, j)),
            pl.BlockSpec((k, bn), lambda i, j: (0, j)),
            pl.BlockSpec((8, bn), lambda i, j: (0, j)),
        ],
        out_specs=[
            pl.BlockSpec((bm, bn), lambda i, j: (i, j)),
            pl.BlockSpec((bm, bn), lambda i, j: (i, j)),
        ],
        out_shape=[jax.ShapeDtypeStruct((np_, hcp), jnp.float32)] * 2,
    )(h, wl, bl8, wr, br8)


def _sc_plan(hcp):
    if hcp >= 4096:
        return 8, 8, 1
    if hcp >= 2048:
        return 16, 8, 2
    if hcp >= 1024:
        return 24, 32, 2
    if hcp >= 512:
        return 48, 64, 2
    return 128, 64, 2


@functools.lru_cache(maxsize=None)
def _make_edge_kernel(hcp, cp, last):
    npc, eb, nslots = _sc_plan(hcp)
    nchunks = -(-N_NODES_S // npc)
    ncp8 = -(-(nchunks + 17) // 8) * 8
    cpw = -(-nchunks // NW)
    cpv = cp // 16
    hpv = hcp // 16
    uu = 16
    while cpv % uu:
        uu //= 2
    uz = 8
    while hpv % uz:
        uz //= 2
    mesh = plsc.VectorSubcoreMesh(core_axis_name="c", subcore_axis_name="s")

    def body(xl_hbm, xr_hbm, att_hbm, bias_hbm, src_hbm, dst_hbm, rptr_hbm,
             out_hbm, xrbuf, numbuf, denbuf, xlbuf0, xlbuf1, srcbuf0, srcbuf1,
             dstbuf0, dstbuf1, attbuf, biasbuf, rptr, hbuf, redscr,
             sem0, sem1):
        XL = (xlbuf0, xlbuf1)
        SRC = (srcbuf0, srcbuf1)
        DST = (dstbuf0, dstbuf1)
        SEM = (sem0, sem1)
        wid = lax.axis_index("s") * 2 + lax.axis_index("c")
        pltpu.sync_copy(att_hbm, attbuf)
        pltpu.sync_copy(bias_hbm, biasbuf)
        pltpu.sync_copy(rptr_hbm, rptr)
        c0 = jnp.minimum(wid * cpw, nchunks)
        c1 = jnp.minimum(c0 + cpw, nchunks)

        zero16 = jnp.zeros((16,), jnp.float32)

        def chunk_body(c, carry):
            base = c * npc
            pltpu.sync_copy(xr_hbm.at[pl.ds(base, npc)], xrbuf)

            def zrow(r, cr):
                @plsc.parallel_loop(0, hpv, step=uz)
                def _zc(j):
                    for u in range(uz):
                        numbuf[r, pl.ds((j + u) * 16, 16)] = zero16
                denbuf[r, :] = zero16
                return cr
            lax.fori_loop(0, npc, zrow, 0)

            rv = rptr[pl.ds(c, 16)]
            e0 = rv[0]
            e1 = rv[1]
            b0 = e0 & ~jnp.int32(7)
            nb = lax.shift_right_arithmetic(
                e1 - b0 + (eb - 1), eb.bit_length() - 1)

            def issue(slot, b):
                bsi = pl.multiple_of(b0 + b * eb, 8)
                pltpu.sync_copy(src_hbm.at[pl.ds(bsi, eb)], SRC[slot])
                pltpu.sync_copy(dst_hbm.at[pl.ds(bsi, eb)],
                                DST[slot].at[pl.ds(0, eb)])
                pltpu.async_copy(xl_hbm.at[SRC[slot]], XL[slot], SEM[slot])

            def drain(slot):
                pltpu.make_async_copy(
                    xl_hbm.at[SRC[slot]], XL[slot], SEM[slot]).wait()

            def process(slot, b):
                bs = pl.multiple_of(b0 + b * eb, 8)
                xlb = XL[slot]
                dstb = DST[slot]

                def edge_body(i, cr2):
                    e = bs + i
                    dv = dstb[pl.ds(i, 16)]
                    dl = jnp.clip(dv[0] - base, 0, npc - 1)
                    inr = jnp.logical_and(e >= e0, e < e1)
                    for h in range(HEADS):
                        o = h * cp

                        @plsc.parallel_loop(
                            0, cpv, step=uu,
                            carry=tuple(zero16 for _ in range(uu)))
                        def accs(j, accs_c):
                            outs = []
                            for u in range(uu):
                                s = pl.ds(o + (j + u) * 16, 16)
                                t = xlb[i, s] + xrbuf[dl, s]
                                t = jnp.maximum(t, 0.2 * t)
                                outs.append(accs_c[u] + t * attbuf[s])
                            return tuple(outs)
                        accl = list(accs)
                        while len(accl) > 1:
                            accl = [accl[q] + accl[q + 1]
                                    for q in range(0, len(accl), 2)]
                        acc = accl[0]
                        lanes = [acc[q] for q in range(16)]
                        while len(lanes) > 1:
                            lanes = [lanes[q] + lanes[q + 1]
                                     for q in range(0, len(lanes), 2)]
                        alpha = jnp.where(inr, lanes[0], -jnp.inf)
                        exv = jnp.exp(jnp.full((16,), alpha, jnp.float32))

                        @plsc.parallel_loop(0, cpv, step=uu)
                        def _wv(j):
                            for u in range(uu):
                                s = pl.ds(o + (j + u) * 16, 16)
                                numbuf[dl, s] = (
                                    numbuf[dl, s] + exv * xlb[i, s])
                        onehot = jnp.where(
                            lax.iota(jnp.int32, 16) == h, 1.0, 0.0)
                        denbuf[dl, :] = denbuf[dl, :] + exv * onehot
                    return cr2
                lax.fori_loop(0, eb, edge_body, 0)

            if nslots == 2:
                @pl.when(nb > 0)
                def _prime():
                    issue(0, jnp.int32(0))

                def pair_body(b2, cr):
                    b = 2 * b2

                    @pl.when(b + 1 < nb)
                    def _i1():
                        issue(1, b + 1)
                    drain(0)
                    process(0, b)

                    @pl.when(b + 1 < nb)
                    def _odd():
                        @pl.when(b + 2 < nb)
                        def _i0():
                            issue(0, b + 2)
                        drain(1)
                        process(1, b + 1)
                    return cr
                nb2 = lax.shift_right_arithmetic(nb + 1, 1)
                lax.fori_loop(0, nb2, pair_body, 0)
            else:
                def batch_body(b, cr):
                    issue(0, b)
                    drain(0)
                    process(0, b)
                    return cr
                lax.fori_loop(0, nb, batch_body, 0)

            def erow(r, cr):
                invv = 1.0 / ((denbuf[r, :] + 1e-16) * HEADS)
                fs = [invv[h] for h in range(HEADS)]

                @plsc.parallel_loop(0, cpv, step=uu)
                def _ec(j):
                    for u in range(uu):
                        s = pl.ds((j + u) * 16, 16)
                        acc = biasbuf[s]
                        for h in range(HEADS):
                            acc = acc + (
                                numbuf[r, pl.ds(h * cp + (j + u) * 16, 16)]
                                * fs[h])
                        if last:
                            acc = 1.0 / (1.0 + jnp.exp(-acc))
                        else:
                            acc = jnp.maximum(acc, 0.0)
                        hbuf[r, s] = acc
                return cr
            lax.fori_loop(0, npc, erow, 0)
            pltpu.sync_copy(hbuf, out_hbm.at[pl.ds(base, npc)])
            return carry
        lax.fori_loop(c0, c1, chunk_body, 0)

    return pl.kernel(
        body,
        out_type=jax.ShapeDtypeStruct((N_PAD, cp), jnp.float32),
        mesh=mesh,
        scratch_types=[
            pltpu.VMEM((npc, hcp), jnp.float32),
            pltpu.VMEM((npc, hcp), jnp.float32),
            pltpu.VMEM((npc, 16), jnp.float32),
            pltpu.VMEM((eb, hcp), jnp.float32),
            pltpu.VMEM((eb, hcp) if nslots == 2 else (8, 16),
                       jnp.float32),
            pltpu.VMEM((eb,), jnp.int32),
            pltpu.VMEM((eb,), jnp.int32),
            pltpu.VMEM((eb + 16,), jnp.int32),
            pltpu.VMEM((eb + 16,), jnp.int32),
            pltpu.VMEM((hcp,), jnp.float32),
            pltpu.VMEM((cp,), jnp.float32),
            pltpu.VMEM((ncp8,), jnp.int32),
            pltpu.VMEM((npc, cp), jnp.float32),
            pltpu.VMEM((HEADS * 16,), jnp.float32),
            pltpu.SemaphoreType.DMA,
            pltpu.SemaphoreType.DMA,
        ],
    )


def _pad_cp(c):
    return max(32, -(-c // 16) * 16)


def kernel(x, edge_index, params):
    n = x.shape[0]
    src = edge_index[0].astype(jnp.int32)
    dst = edge_index[1].astype(jnp.int32)
    order = jnp.argsort(dst)
    sdst = dst[order]
    ssrc = src[order]
    epad = 64
    ssrc_p = jnp.concatenate([ssrc, jnp.zeros((epad,), jnp.int32)])
    sdst_p = jnp.concatenate([sdst, jnp.zeros((epad,), jnp.int32)])

    rptrs = {}
    for p in params:
        hh, c = p["att"].shape
        cp = _pad_cp(c)
        npc, _, _ = _sc_plan(hh * cp)
        if npc not in rptrs:
            nchunks = -(-N_NODES_S // npc)
            ncp8 = -(-(nchunks + 17) // 8) * 8
            bounds = jnp.arange(nchunks + 1, dtype=jnp.int32) * npc
            rp = jnp.searchsorted(sdst, bounds, side="left").astype(jnp.int32)
            rp = jnp.concatenate(
                [rp, jnp.zeros((ncp8 - (nchunks + 1),), jnp.int32)])
            rptrs[npc] = rp

    h = jnp.zeros((N_PAD, 128), jnp.float32)
    h = h.at[:n, : x.shape[1]].set(x)

    for li, p in enumerate(params):
        hh, c = p["att"].shape
        cin = p["W_l"].shape[0]
        cp = _pad_cp(c)
        hcp = hh * cp
        kin = h.shape[1]

        def expand_w(w):
            w3 = jnp.zeros((kin, hh, cp), jnp.float32)
            w3 = w3.at[:cin, :, :c].set(w.reshape(cin, hh, c))
            return w3.reshape(kin, hcp)

        def expand_b(b):
            b3 = jnp.zeros((hh, cp), jnp.float32)
            b3 = b3.at[:, :c].set(b.reshape(hh, c))
            return b3.reshape(hcp)

        wl = expand_w(p["W_l"])
        wr = expand_w(p["W_r"])
        bl8 = jnp.tile(expand_b(p["b_l"]).reshape(1, -1), (8, 1))
        br8 = jnp.tile(expand_b(p["b_r"]).reshape(1, -1), (8, 1))
        attf = jnp.zeros((hh, cp), jnp.float32).at[:, :c].set(
            p["att"]).reshape(hcp)
        biasf = jnp.zeros((cp,), jnp.float32).at[:c].set(p["bias"])

        xl, xr = _dual_matmul(h, wl, bl8, wr, br8)

        npc, _, _ = _sc_plan(hcp)
        ek = _make_edge_kernel(hcp, cp, li == len(params) - 1)
        h = ek(xl, xr, attf, biasf, ssrc_p, sdst_p, rptrs[npc])
        if c >= 128:
            h = h[:, :c]

    return h[:n, :1]


# speedup vs baseline: 1.4824x; 1.4824x over previous
import functools

import jax
import jax.numpy as jnp
from jax import lax
from jax.experimental import pallas as pl
from jax.experimental.pallas import tpu as pltpu
from jax.experimental.pallas import tpu_sc as plsc

N_NODES_S = 5000
N_PAD = 5120
N_EDGES_S = 20000
HEADS = 4
NW = 32


def _mm2_body(h_ref, wl_ref, bl_ref, wr_ref, br_ref, xl_ref, xr_ref):
    h = h_ref[...]
    xl_ref[...] = (
        jnp.dot(h, wl_ref[...], preferred_element_type=jnp.float32)
        + bl_ref[0:1, :]
    )
    xr_ref[...] = (
        jnp.dot(h, wr_ref[...], preferred_element_type=jnp.float32)
        + br_ref[0:1, :]
    )


@functools.partial(jax.jit, static_argnames=("bm", "bn"))
def _dual_matmul(h, wl, bl8, wr, br8, bm=512, bn=512):
    np_, k = h.shape
    hcp = wl.shape[1]
    bn = min(bn, hcp)
    grid = (np_ // bm, hcp // bn)
    return pl.pallas_call(
        _mm2_body,
        grid=grid,
        in_specs=[
            pl.BlockSpec((bm, k), lambda i, j: (i, 0)),
            pl.BlockSpec((k, bn), lambda i, j: (0, j)),
            pl.BlockSpec((8, bn), lambda i, j: (0, j)),
            pl.BlockSpec((k, bn), lambda i, j: (0, j)),
            pl.BlockSpec((8, bn), lambda i, j: (0, j)),
        ],
        out_specs=[
            pl.BlockSpec((bm, bn), lambda i, j: (i, j)),
            pl.BlockSpec((bm, bn), lambda i, j: (i, j)),
        ],
        out_shape=[jax.ShapeDtypeStruct((np_, hcp), jnp.float32)] * 2,
    )(h, wl, bl8, wr, br8)


def _sc_plan(hcp):
    if hcp >= 4096:
        return 8, 8, 1
    if hcp >= 2048:
        return 16, 8, 2
    if hcp >= 1024:
        return 24, 32, 2
    if hcp >= 512:
        return 48, 64, 2
    return 128, 64, 2


@functools.lru_cache(maxsize=None)
def _make_edge_kernel(hcp, cp, last):
    npc, eb, nslots = _sc_plan(hcp)
    nchunks = -(-N_NODES_S // npc)
    ncp8 = -(-(nchunks + 17) // 8) * 8
    cpw = -(-nchunks // NW)
    cpv = cp // 16
    hpv = hcp // 16
    uu = 8
    while cpv % uu:
        uu //= 2
    uz = 8
    while hpv % uz:
        uz //= 2
    mesh = plsc.VectorSubcoreMesh(core_axis_name="c", subcore_axis_name="s")

    def body(xl_hbm, xr_hbm, att_hbm, bias_hbm, src_hbm, dst_hbm, rptr_hbm,
             out_hbm, xrbuf, numbuf, denbuf, xlbuf0, xlbuf1, srcbuf0, srcbuf1,
             dstbuf0, dstbuf1, attbuf, biasbuf, rptr, hbuf, redscr,
             sem0, sem1):
        XL = (xlbuf0, xlbuf1)
        SRC = (srcbuf0, srcbuf1)
        DST = (dstbuf0, dstbuf1)
        SEM = (sem0, sem1)
        wid = lax.axis_index("s") * 2 + lax.axis_index("c")
        pltpu.sync_copy(att_hbm, attbuf)
        pltpu.sync_copy(bias_hbm, biasbuf)
        pltpu.sync_copy(rptr_hbm, rptr)
        c0 = jnp.minimum(wid * cpw, nchunks)
        c1 = jnp.minimum(c0 + cpw, nchunks)

        zero16 = jnp.zeros((16,), jnp.float32)

        def chunk_body(c, carry):
            base = c * npc
            pltpu.sync_copy(xr_hbm.at[pl.ds(base, npc)], xrbuf)

            def zrow(r, cr):
                @plsc.parallel_loop(0, hpv, step=uz)
                def _zc(j):
                    for u in range(uz):
                        numbuf[r, pl.ds((j + u) * 16, 16)] = zero16
                denbuf[r, :] = zero16
                return cr
            lax.fori_loop(0, npc, zrow, 0)

            rv = rptr[pl.ds(c, 16)]
            e0 = rv[0]
            e1 = rv[1]
            b0 = e0 & ~jnp.int32(7)
            nb = lax.shift_right_arithmetic(
                e1 - b0 + (eb - 1), eb.bit_length() - 1)

            def issue(slot, b):
                bsi = pl.multiple_of(b0 + b * eb, 8)
                pltpu.sync_copy(src_hbm.at[pl.ds(bsi, eb)], SRC[slot])
                pltpu.sync_copy(dst_hbm.at[pl.ds(bsi, eb)],
                                DST[slot].at[pl.ds(0, eb)])
                pltpu.async_copy(xl_hbm.at[SRC[slot]], XL[slot], SEM[slot])

            def drain(slot):
                pltpu.make_async_copy(
                    xl_hbm.at[SRC[slot]], XL[slot], SEM[slot]).wait()

            def process(slot, b):
                bs = pl.multiple_of(b0 + b * eb, 8)
                xlb = XL[slot]
                dstb = DST[slot]

                def edge_body(i, cr2):
                    e = bs + i
                    dv = dstb[pl.ds(i, 16)]
                    dl = jnp.clip(dv[0] - base, 0, npc - 1)
                    inr = jnp.logical_and(e >= e0, e < e1)
                    head_accs = []
                    for h in range(HEADS):
                        o = h * cp

                        @plsc.parallel_loop(
                            0, cpv, step=uu,
                            carry=tuple(zero16 for _ in range(uu)))
                        def accs(j, accs_c):
                            outs = []
                            for u in range(uu):
                                s = pl.ds(o + (j + u) * 16, 16)
                                t = xlb[i, s] + xrbuf[dl, s]
                                t = jnp.maximum(t, 0.2 * t)
                                outs.append(accs_c[u] + t * attbuf[s])
                            return tuple(outs)
                        accl = list(accs)
                        while len(accl) > 1:
                            accl = [accl[q] + accl[q + 1]
                                    for q in range(0, len(accl), 2)]
                        head_accs.append(accl[0])
                    exvs = []
                    dencomb = zero16
                    for h in range(HEADS):
                        lanes = [head_accs[h][q] for q in range(16)]
                        while len(lanes) > 1:
                            lanes = [lanes[q] + lanes[q + 1]
                                     for q in range(0, len(lanes), 2)]
                        alpha = jnp.where(inr, lanes[0], -jnp.inf)
                        exv = jnp.exp(jnp.full((16,), alpha, jnp.float32))
                        exvs.append(exv)
                        onehot = jnp.where(
                            lax.iota(jnp.int32, 16) == h, 1.0, 0.0)
                        dencomb = dencomb + exv * onehot
                    denbuf[dl, :] = denbuf[dl, :] + dencomb
                    for h in range(HEADS):
                        o = h * cp
                        exv = exvs[h]

                        @plsc.parallel_loop(0, cpv, step=uu)
                        def _wv(j):
                            for u in range(uu):
                                s = pl.ds(o + (j + u) * 16, 16)
                                numbuf[dl, s] = (
                                    numbuf[dl, s] + exv * xlb[i, s])
                    return cr2
                lax.fori_loop(0, eb, edge_body, 0)

            if nslots == 2:
                @pl.when(nb > 0)
                def _prime():
                    issue(0, jnp.int32(0))

                def pair_body(b2, cr):
                    b = 2 * b2

                    @pl.when(b + 1 < nb)
                    def _i1():
                        issue(1, b + 1)
                    drain(0)
                    process(0, b)

                    @pl.when(b + 1 < nb)
                    def _odd():
                        @pl.when(b + 2 < nb)
                        def _i0():
                            issue(0, b + 2)
                        drain(1)
                        process(1, b + 1)
                    return cr
                nb2 = lax.shift_right_arithmetic(nb + 1, 1)
                lax.fori_loop(0, nb2, pair_body, 0)
            else:
                def batch_body(b, cr):
                    issue(0, b)
                    drain(0)
                    process(0, b)
                    return cr
                lax.fori_loop(0, nb, batch_body, 0)

            def erow(r, cr):
                invv = 1.0 / ((denbuf[r, :] + 1e-16) * HEADS)
                fs = [invv[h] for h in range(HEADS)]

                @plsc.parallel_loop(0, cpv, step=uu)
                def _ec(j):
                    for u in range(uu):
                        s = pl.ds((j + u) * 16, 16)
                        acc = biasbuf[s]
                        for h in range(HEADS):
                            acc = acc + (
                                numbuf[r, pl.ds(h * cp + (j + u) * 16, 16)]
                                * fs[h])
                        if last:
                            acc = 1.0 / (1.0 + jnp.exp(-acc))
                        else:
                            acc = jnp.maximum(acc, 0.0)
                        hbuf[r, s] = acc
                return cr
            lax.fori_loop(0, npc, erow, 0)
            pltpu.sync_copy(hbuf, out_hbm.at[pl.ds(base, npc)])
            return carry
        lax.fori_loop(c0, c1, chunk_body, 0)

    return pl.kernel(
        body,
        out_type=jax.ShapeDtypeStruct((N_PAD, cp), jnp.float32),
        mesh=mesh,
        scratch_types=[
            pltpu.VMEM((npc, hcp), jnp.float32),
            pltpu.VMEM((npc, hcp), jnp.float32),
            pltpu.VMEM((npc, 16), jnp.float32),
            pltpu.VMEM((eb, hcp), jnp.float32),
            pltpu.VMEM((eb, hcp) if nslots == 2 else (8, 16),
                       jnp.float32),
            pltpu.VMEM((eb,), jnp.int32),
            pltpu.VMEM((eb,), jnp.int32),
            pltpu.VMEM((eb + 16,), jnp.int32),
            pltpu.VMEM((eb + 16,), jnp.int32),
            pltpu.VMEM((hcp,), jnp.float32),
            pltpu.VMEM((cp,), jnp.float32),
            pltpu.VMEM((ncp8,), jnp.int32),
            pltpu.VMEM((npc, cp), jnp.float32),
            pltpu.VMEM((HEADS * 16,), jnp.float32),
            pltpu.SemaphoreType.DMA,
            pltpu.SemaphoreType.DMA,
        ],
    )


def _pad_cp(c):
    return max(32, -(-c // 16) * 16)


def kernel(x, edge_index, params):
    n = x.shape[0]
    src = edge_index[0].astype(jnp.int32)
    dst = edge_index[1].astype(jnp.int32)
    order = jnp.argsort(dst)
    sdst = dst[order]
    ssrc = src[order]
    epad = 64
    ssrc_p = jnp.concatenate([ssrc, jnp.zeros((epad,), jnp.int32)])
    sdst_p = jnp.concatenate([sdst, jnp.zeros((epad,), jnp.int32)])

    rptrs = {}
    for p in params:
        hh, c = p["att"].shape
        cp = _pad_cp(c)
        npc, _, _ = _sc_plan(hh * cp)
        if npc not in rptrs:
            nchunks = -(-N_NODES_S // npc)
            ncp8 = -(-(nchunks + 17) // 8) * 8
            bounds = jnp.arange(nchunks + 1, dtype=jnp.int32) * npc
            rp = jnp.searchsorted(sdst, bounds, side="left").astype(jnp.int32)
            rp = jnp.concatenate(
                [rp, jnp.zeros((ncp8 - (nchunks + 1),), jnp.int32)])
            rptrs[npc] = rp

    h = jnp.zeros((N_PAD, 128), jnp.float32)
    h = h.at[:n, : x.shape[1]].set(x)

    for li, p in enumerate(params):
        hh, c = p["att"].shape
        cin = p["W_l"].shape[0]
        cp = _pad_cp(c)
        hcp = hh * cp
        kin = h.shape[1]

        def expand_w(w):
            w3 = jnp.zeros((kin, hh, cp), jnp.float32)
            w3 = w3.at[:cin, :, :c].set(w.reshape(cin, hh, c))
            return w3.reshape(kin, hcp)

        def expand_b(b):
            b3 = jnp.zeros((hh, cp), jnp.float32)
            b3 = b3.at[:, :c].set(b.reshape(hh, c))
            return b3.reshape(hcp)

        wl = expand_w(p["W_l"])
        wr = expand_w(p["W_r"])
        bl8 = jnp.tile(expand_b(p["b_l"]).reshape(1, -1), (8, 1))
        br8 = jnp.tile(expand_b(p["b_r"]).reshape(1, -1), (8, 1))
        attf = jnp.zeros((hh, cp), jnp.float32).at[:, :c].set(
            p["att"]).reshape(hcp)
        biasf = jnp.zeros((cp,), jnp.float32).at[:c].set(p["bias"])

        xl, xr = _dual_matmul(h, wl, bl8, wr, br8)

        npc, _, _ = _sc_plan(hcp)
        ek = _make_edge_kernel(hcp, cp, li == len(params) - 1)
        h = ek(xl, xr, attf, biasf, ssrc_p, sdst_p, rptrs[npc])
        if c >= 128:
            h = h[:, :c]

    return h[:n, :1]
